# baseline (speedup 1.0000x reference)
import functools

import jax
import jax.numpy as jnp
from jax.experimental import pallas as pl
from jax.experimental.pallas import tpu as pltpu
from jax.experimental.pallas import tpu_sc as plsc

N = 10000
K = 16
NPAD = 10240
TILE = 512
NTILES = NPAD // TILE
CHUNKS = 2
CS = NPAD // CHUNKS
GATHER_WINDOW = 256
GW = 128
GDTYPE = jnp.float32
PREC = jax.lax.Precision.DEFAULT
PREC_HI = jax.lax.Precision.HIGHEST


def _sc_gather(table, idx):
    num_idx = idx.shape[1]
    c = table.shape[1]
    mesh = plsc.VectorSubcoreMesh(core_axis_name="core", subcore_axis_name="subcore")

    window = 256 if c <= 128 else 128 if c <= 256 else 80 if c <= 384 else 64

    @pl.kernel(
        out_type=jax.ShapeDtypeStruct((num_idx, c), table.dtype),
        mesh=mesh,
    )
    def gather_kernel(x_hbm, i_hbm, o_hbm):
        def body(i_vmem, o_vmem):
            pltpu.sync_copy(x_hbm.at[i_vmem.at[0]], o_vmem)

        pltpu.emit_pipeline(
            body,
            grid=(num_idx // window,),
            in_specs=[pl.BlockSpec((1, window), index_map=lambda i: (0, i))],
            out_specs=[pl.BlockSpec((window, c), index_map=lambda i: (i, 0))],
            core_axis_name=("core", "subcore"),
            dimension_semantics=(pltpu.PARALLEL,),
        )(i_hbm, o_hbm)

    return gather_kernel(table, idx)


def _pairs_acc(refs, pos, n_pairs, mul_first):
    if mul_first:
        x0 = refs[pos][...] * refs[pos + 1][...]
        acc = jnp.dot(x0, refs[pos + 2][...], precision=PREC,
                      preferred_element_type=jnp.float32)
        pos += 3
    else:
        acc = jnp.dot(refs[pos][...], refs[pos + 1][...], precision=PREC,
                      preferred_element_type=jnp.float32)
        pos += 2
    for _ in range(1, n_pairs):
        acc = acc + jnp.dot(refs[pos][...], refs[pos + 1][...], precision=PREC,
                            preferred_element_type=jnp.float32)
        pos += 2
    return acc, pos


def _a_body(*refs, n_pairs, mul_first, cout):
    pc_ref, w1r_ref, b1_ref = refs[2 * n_pairs + (1 if mul_first else 0):-2]
    a_ref, c_ref = refs[-2:]
    p = jnp.dot(pc_ref[...], w1r_ref[...], precision=PREC,
                preferred_element_type=jnp.float32)
    acc, _ = _pairs_acc(refs, 0, n_pairs, mul_first)
    a = acc + p
    if cout < GW:
        a = jnp.pad(a, ((0, 0), (0, GW - cout)))
    a_ref[...] = a.astype(a_ref.dtype)
    c_ref[...] = b1_ref[...] - p


def _a_kernel(pairs, pc16, w1r16, b1, mul_first=False):
    cout = b1.shape[1]
    in_specs = []
    args = []
    n_pairs = len(pairs)
    for tup in pairs:
        for arr in tup:
            if arr.shape[0] == NPAD:
                in_specs.append(
                    pl.BlockSpec((TILE, arr.shape[1]), lambda i: (i, 0)))
            else:
                in_specs.append(pl.BlockSpec(arr.shape, lambda i: (0, 0)))
            args.append(arr)
    in_specs.append(pl.BlockSpec((TILE, 16), lambda i: (i, 0)))
    args.append(pc16)
    for arr in (w1r16, b1):
        in_specs.append(pl.BlockSpec(arr.shape, lambda i: (0, 0)))
        args.append(arr)
    return pl.pallas_call(
        functools.partial(_a_body, n_pairs=n_pairs, mul_first=mul_first,
                          cout=cout),
        grid=(NTILES,),
        in_specs=in_specs,
        out_specs=[
            pl.BlockSpec((TILE, GW), lambda i: (i, 0)),
            pl.BlockSpec((TILE, cout), lambda i: (i, 0)),
        ],
        out_shape=[
            jax.ShapeDtypeStruct((NPAD, GW), GDTYPE),
            jax.ShapeDtypeStruct((NPAD, cout), jnp.float32),
        ],
    )(*args)


def _a2_body(*refs, n_a, n_b, cout_a, cout_b):
    pc_ref, w1ra_ref, b1a_ref, w1rb_ref, b1b_ref = refs[2 * (n_a + n_b):-3]
    a_ref, ca_ref, cb_ref = refs[-3:]
    pc = pc_ref[...]
    pa = jnp.dot(pc, w1ra_ref[...], precision=PREC,
                 preferred_element_type=jnp.float32)
    pb = jnp.dot(pc, w1rb_ref[...], precision=PREC,
                 preferred_element_type=jnp.float32)
    acc_a, pos = _pairs_acc(refs, 0, n_a, False)
    acc_b, _ = _pairs_acc(refs, pos, n_b, False)
    aa = acc_a + pa
    ab = acc_b + pb
    if cout_a < GW:
        aa = jnp.pad(aa, ((0, 0), (0, GW - cout_a)))
    if cout_b < GW:
        ab = jnp.pad(ab, ((0, 0), (0, GW - cout_b)))
    bits_a = jax.lax.bitcast_convert_type(
        aa.astype(jnp.bfloat16).astype(jnp.float32), jnp.uint32)
    bits_b = jax.lax.bitcast_convert_type(
        ab.astype(jnp.bfloat16).astype(jnp.float32), jnp.uint32)
    a_ref[...] = (bits_a & jnp.uint32(0xFFFF0000)) | (bits_b >> 16)
    ca_ref[...] = b1a_ref[...] - pa
    cb_ref[...] = b1b_ref[...] - pb


def _a2_kernel(pairs_a, sa, pairs_b, sb, pc16):
    cout_a = sa["b1"].shape[1]
    cout_b = sb["b1"].shape[1]
    in_specs = []
    args = []
    for tup in pairs_a + pairs_b:
        for arr in tup:
            if arr.shape[0] == NPAD:
                in_specs.append(
                    pl.BlockSpec((TILE, arr.shape[1]), lambda i: (i, 0)))
            else:
                in_specs.append(pl.BlockSpec(arr.shape, lambda i: (0, 0)))
            args.append(arr)
    in_specs.append(pl.BlockSpec((TILE, 16), lambda i: (i, 0)))
    args.append(pc16)
    for arr in (sa["w1r"], sa["b1"], sb["w1r"], sb["b1"]):
        in_specs.append(pl.BlockSpec(arr.shape, lambda i: (0, 0)))
        args.append(arr)
    return pl.pallas_call(
        functools.partial(_a2_body, n_a=len(pairs_a), n_b=len(pairs_b),
                          cout_a=cout_a, cout_b=cout_b),
        grid=(NTILES,),
        in_specs=in_specs,
        out_specs=[
            pl.BlockSpec((TILE, GW), lambda i: (i, 0)),
            pl.BlockSpec((TILE, cout_a), lambda i: (i, 0)),
            pl.BlockSpec((TILE, cout_b), lambda i: (i, 0)),
        ],
        out_shape=[
            jax.ShapeDtypeStruct((NPAD, GW), jnp.uint32),
            jax.ShapeDtypeStruct((NPAD, cout_a), jnp.float32),
            jax.ShapeDtypeStruct((NPAD, cout_b), jnp.float32),
        ],
    )(*args)


def _lrelu(x):
    return jnp.where(x >= 0, x, 0.1 * x)


def _mlp_body(g_ref, c_ref, w2_ref, b2_ref, w3_ref, b3_ref, *rest,
              act, has_res, cout, unpack):
    if has_res:
        res_ref, o_ref = rest
    else:
        (o_ref,) = rest
    c = c_ref[...]
    w2 = w2_ref[...]
    b2 = b2_ref[...]
    w3 = w3_ref[...]
    b3 = b3_ref[...]
    acc = None
    for k in range(K):
        g = g_ref[k]
        if unpack == "hi":
            g = jax.lax.bitcast_convert_type(
                g & jnp.uint32(0xFFFF0000), jnp.float32)
        elif unpack == "lo":
            g = jax.lax.bitcast_convert_type(g << 16, jnp.float32)
        h = _lrelu(g[:, :cout].astype(jnp.float32) + c)
        h = _lrelu(jnp.dot(h, w2, precision=PREC,
                           preferred_element_type=jnp.float32) + b2)
        h = _lrelu(jnp.dot(h, w3, precision=PREC,
                           preferred_element_type=jnp.float32) + b3)
        acc = h if acc is None else jnp.maximum(acc, h)
    if act == "sigmoid":
        acc = jax.nn.sigmoid(acc)
    elif act == "tanh":
        acc = jnp.tanh(acc)
    if has_res:
        acc = acc + res_ref[...]
    o_ref[...] = acc


def _mlp_max(g, cbias, p, act=None, res=None, lane=0, unpack=None):
    cout = cbias.shape[1]
    cout2 = p["w3"].shape[1]
    rows = g.shape[1]
    in_specs = [
        pl.BlockSpec((K, TILE, GW), lambda i, lane=lane: (0, i, lane)),
        pl.BlockSpec((TILE, cout), lambda i: (i, 0)),
        pl.BlockSpec(p["w2"].shape, lambda i: (0, 0)),
        pl.BlockSpec((1, cout2), lambda i: (0, 0)),
        pl.BlockSpec(p["w3"].shape, lambda i: (0, 0)),
        pl.BlockSpec((1, cout2), lambda i: (0, 0)),
    ]
    args = [g, cbias, p["w2"], p["b2"], p["w3"], p["b3"]]
    if res is not None:
        in_specs.append(pl.BlockSpec((TILE, cout2), lambda i: (i, 0)))
        args.append(res)
    return pl.pallas_call(
        functools.partial(_mlp_body, act=act, has_res=res is not None,
                          cout=cout, unpack=unpack),
        grid=(rows // TILE,),
        in_specs=in_specs,
        out_specs=pl.BlockSpec((TILE, cout2), lambda i: (i, 0)),
        out_shape=jax.ShapeDtypeStruct((rows, cout2), jnp.float32),
    )(*args)


def _gru_body(h_ref, z_ref, q_ref, o_ref):
    z = z_ref[...]
    o_ref[...] = (1.0 - z) * h_ref[...] + z * q_ref[...]


def _gru_update(h, z, q):
    d = h.shape[1]
    spec = pl.BlockSpec((TILE, d), lambda i: (i, 0))
    return pl.pallas_call(
        _gru_body,
        grid=(NTILES,),
        in_specs=[spec, spec, spec],
        out_specs=spec,
        out_shape=jax.ShapeDtypeStruct((NPAD, d), jnp.float32),
    )(h, z, q)


def _fc_body(f_ref, w_ref, b_ref, flow_ref, o_ref):
    o_ref[...] = (
        jnp.dot(f_ref[...], w_ref[...], precision=PREC_HI,
                preferred_element_type=jnp.float32)
        + b_ref[...]
        + flow_ref[...]
    )


def _fc_res(f, w, b, flow8):
    cin = f.shape[1]
    cout = w.shape[1]
    return pl.pallas_call(
        _fc_body,
        grid=(NTILES,),
        in_specs=[
            pl.BlockSpec((TILE, cin), lambda i: (i, 0)),
            pl.BlockSpec((cin, cout), lambda i: (0, 0)),
            pl.BlockSpec((1, cout), lambda i: (0, 0)),
            pl.BlockSpec((TILE, cout), lambda i: (i, 0)),
        ],
        out_specs=pl.BlockSpec((TILE, cout), lambda i: (i, 0)),
        out_shape=jax.ShapeDtypeStruct((NPAD, cout), jnp.float32),
    )(f, w, b, flow8)


def _split_setconv(p, cin):
    w1 = p["l1"]["W"]
    cout = w1.shape[1]
    w1r = jnp.zeros((16, cout), jnp.float32).at[:3].set(w1[cin:cin + 3])
    return {
        "w1s": w1[:cin],
        "w1r": w1r,
        "b1": p["l1"]["b"][None, :],
        "w2": p["l2"]["W"],
        "b2": p["l2"]["b"][None, :],
        "w3": p["l3"]["W"],
        "b3": p["l3"]["b"][None, :],
    }


def _pad_rows(x):
    return jnp.pad(x, ((0, NPAD - x.shape[0]), (0, 0)))


def kernel(pc, edges, flow, params):
    pc0 = pc[0]
    flow0 = flow[0]
    e = edges[0].astype(jnp.int32)

    et = jnp.pad(e.T, ((0, 0), (0, NPAD - N)))
    idxs = [et[:, c * CS:(c + 1) * CS].reshape(1, K * CS) for c in range(CHUNKS)]

    pc16 = jnp.pad(_pad_rows(pc0), ((0, 0), (0, 13)))
    flow8 = jnp.pad(_pad_rows(flow0), ((0, 0), (0, 5)))

    sp = {name: _split_setconv(params[name], cin) for name, cin in [
        ("feat_conv1", 3), ("feat_conv2", 32), ("feat_conv3", 64),
        ("h_conv1", 128), ("h_conv2", 128), ("delta_flow_conv_x", 128),
        ("flow_conv_x", 128), ("convz", 256), ("convr", 256), ("convq", 256),
        ("flow_conv1", 128), ("flow_conv2", 128),
    ]}

    def make_table(pairs, name, mul_first=False):
        s = sp[name]
        return _a_kernel(pairs, pc16, s["w1r"], s["b1"], mul_first=mul_first)

    def gather(table):
        return [_sc_gather(table, idxs[c]).reshape(K, CS, table.shape[1])
                for c in range(CHUNKS)]

    def mlp(gchunks, cb, name, act=None, res=None, unpack=None):
        outs = []
        for c, g in enumerate(gchunks):
            sl = slice(c * CS, (c + 1) * CS)
            outs.append(_mlp_max(
                g, cb[sl], sp[name], act=act,
                res=None if res is None else res[sl], unpack=unpack))
        return jnp.concatenate(outs, axis=0)

    def layer(pairs, name, act=None, res=None, mul_first=False):
        a, cb = make_table(pairs, name, mul_first=mul_first)
        return mlp(gather(a), cb, name, act=act, res=res)

    we = params["flow_encoder"]["W"]
    be = params["flow_encoder"]["b"]
    wfold = jnp.zeros((8, 128), jnp.float32).at[:3].set(
        jnp.dot(we, sp["flow_conv_x"]["w1s"], precision=PREC_HI))
    sp["flow_conv_x"]["b1"] = sp["flow_conv_x"]["b1"] + jnp.dot(
        be, sp["flow_conv_x"]["w1s"], precision=PREC_HI)[None, :]

    w1s_fc1_16 = jnp.zeros((16, 32), jnp.float32).at[:3].set(sp["feat_conv1"]["w1s"])
    a1, cb_fc1, cb_fx = _a2_kernel(
        [(pc16, w1s_fc1_16)], sp["feat_conv1"],
        [(flow8, wfold)], sp["flow_conv_x"], pc16)
    g1 = gather(a1)
    feat = mlp(g1, cb_fc1, "feat_conv1", unpack="hi")
    feat = layer([(feat, sp["feat_conv2"]["w1s"])], "feat_conv2")
    feat = layer([(feat, sp["feat_conv3"]["w1s"])], "feat_conv3")

    a4, cb_h1, cb_df = _a2_kernel(
        [(feat, sp["h_conv1"]["w1s"])], sp["h_conv1"],
        [(feat, sp["delta_flow_conv_x"]["w1s"])], sp["delta_flow_conv_x"], pc16)
    g4 = gather(a4)
    h = mlp(g4, cb_h1, "h_conv1", unpack="hi")
    corr = mlp(g4, cb_df, "delta_flow_conv_x", unpack="lo")
    h = layer([(h, sp["h_conv2"]["w1s"])], "h_conv2", act="tanh")
    x = mlp(g1, cb_fx, "flow_conv_x", res=corr, unpack="lo")

    a6, cb_z, cb_r = _a2_kernel(
        [(h, sp["convz"]["w1s"][:128]), (x, sp["convz"]["w1s"][128:])],
        sp["convz"],
        [(h, sp["convr"]["w1s"][:128]), (x, sp["convr"]["w1s"][128:])],
        sp["convr"], pc16)
    g6 = gather(a6)
    z = mlp(g6, cb_z, "convz", act="sigmoid", unpack="hi")
    r = mlp(g6, cb_r, "convr", act="sigmoid", unpack="lo")
    q = layer([(r, h, sp["convq"]["w1s"][:128]), (x, sp["convq"]["w1s"][128:])],
              "convq", act="tanh", mul_first=True)
    h = _gru_update(h, z, q)

    f = layer([(h, sp["flow_conv1"]["w1s"])], "flow_conv1")
    f = layer([(f, sp["flow_conv2"]["w1s"])], "flow_conv2")
    fcw = jnp.pad(params["fc"]["W"], ((0, 0), (0, 5)))
    fcb = jnp.pad(params["fc"]["b"], (0, 5))[None, :]
    out = _fc_res(f, fcw, fcb, flow8)

    return out[:N, :3][None]


# speedup vs baseline: 1.0209x; 1.0209x over previous
import functools

import jax
import jax.numpy as jnp
from jax.experimental import pallas as pl
from jax.experimental.pallas import tpu as pltpu
from jax.experimental.pallas import tpu_sc as plsc

N = 10000
K = 16
NPAD = 10240
TILE = 512
NTILES = NPAD // TILE
CHUNKS = 1
CS = NPAD // CHUNKS
GATHER_WINDOW = 256
GW = 128
GDTYPE = jnp.float32
PREC = jax.lax.Precision.DEFAULT
PREC_HI = jax.lax.Precision.HIGHEST


def _sc_gather(table, idx):
    num_idx = idx.shape[1]
    c = table.shape[1]
    mesh = plsc.VectorSubcoreMesh(core_axis_name="core", subcore_axis_name="subcore")

    window = 256 if c <= 128 else 128 if c <= 256 else 80 if c <= 384 else 64

    per_unit = num_idx // 32
    nwin = per_unit // window

    @pl.kernel(
        out_type=jax.ShapeDtypeStruct((num_idx, c), table.dtype),
        mesh=mesh,
        scratch_types=[
            pltpu.VMEM((1, per_unit), jnp.int32),
            pltpu.VMEM((window, c), table.dtype),
            pltpu.VMEM((window, c), table.dtype),
            pltpu.VMEM((window, c), table.dtype),
            pltpu.SemaphoreType.DMA,
            pltpu.SemaphoreType.DMA,
            pltpu.SemaphoreType.DMA,
            pltpu.SemaphoreType.DMA,
            pltpu.SemaphoreType.DMA,
            pltpu.SemaphoreType.DMA,
            pltpu.SemaphoreType.DMA,
        ],
    )
    def gather_kernel(x_hbm, i_hbm, o_hbm, ibuf, b0, b1, b2,
                      isem, g0, g1, g2, o0, o1, o2):
        core = jax.lax.axis_index("core")
        sub = jax.lax.axis_index("subcore")
        u = core * 16 + sub
        base = u * per_unit
        pltpu.async_copy(i_hbm.at[:, pl.ds(base, per_unit)], ibuf, isem).wait()
        bufs = (b0, b1, b2)
        gsems = (g0, g1, g2)
        osems = (o0, o1, o2)
        ghandles = [None] * nwin
        ohandles = [None] * nwin
        for w in range(nwin):
            b = w % 3
            if w >= 3:
                ohandles[w - 3].wait()
            ghandles[w] = pltpu.async_copy(
                x_hbm.at[ibuf.at[0, pl.ds(w * window, window)]],
                bufs[b], gsems[b])
            if w >= 1:
                ghandles[w - 1].wait()
                ohandles[w - 1] = pltpu.async_copy(
                    bufs[(w - 1) % 3],
                    o_hbm.at[pl.ds(base + (w - 1) * window, window)],
                    osems[(w - 1) % 3])
        ghandles[nwin - 1].wait()
        ohandles[nwin - 1] = pltpu.async_copy(
            bufs[(nwin - 1) % 3],
            o_hbm.at[pl.ds(base + (nwin - 1) * window, window)],
            osems[(nwin - 1) % 3])
        ohandles[nwin - 2].wait()
        ohandles[nwin - 1].wait()

    return gather_kernel(table, idx)


def _pairs_acc(refs, pos, n_pairs, mul_first):
    if mul_first:
        x0 = refs[pos][...] * refs[pos + 1][...]
        acc = jnp.dot(x0, refs[pos + 2][...], precision=PREC,
                      preferred_element_type=jnp.float32)
        pos += 3
    else:
        acc = jnp.dot(refs[pos][...], refs[pos + 1][...], precision=PREC,
                      preferred_element_type=jnp.float32)
        pos += 2
    for _ in range(1, n_pairs):
        acc = acc + jnp.dot(refs[pos][...], refs[pos + 1][...], precision=PREC,
                            preferred_element_type=jnp.float32)
        pos += 2
    return acc, pos


def _a_body(*refs, n_pairs, mul_first, cout):
    pc_ref, w1r_ref, b1_ref = refs[2 * n_pairs + (1 if mul_first else 0):-2]
    a_ref, c_ref = refs[-2:]
    p = jnp.dot(pc_ref[...], w1r_ref[...], precision=PREC,
                preferred_element_type=jnp.float32)
    acc, _ = _pairs_acc(refs, 0, n_pairs, mul_first)
    a = acc + p
    if cout < GW:
        a = jnp.pad(a, ((0, 0), (0, GW - cout)))
    a_ref[...] = a.astype(a_ref.dtype)
    c_ref[...] = b1_ref[...] - p


def _a_kernel(pairs, pc16, w1r16, b1, mul_first=False):
    cout = b1.shape[1]
    in_specs = []
    args = []
    n_pairs = len(pairs)
    for tup in pairs:
        for arr in tup:
            if arr.shape[0] == NPAD:
                in_specs.append(
                    pl.BlockSpec((TILE, arr.shape[1]), lambda i: (i, 0)))
            else:
                in_specs.append(pl.BlockSpec(arr.shape, lambda i: (0, 0)))
            args.append(arr)
    in_specs.append(pl.BlockSpec((TILE, 16), lambda i: (i, 0)))
    args.append(pc16)
    for arr in (w1r16, b1):
        in_specs.append(pl.BlockSpec(arr.shape, lambda i: (0, 0)))
        args.append(arr)
    return pl.pallas_call(
        functools.partial(_a_body, n_pairs=n_pairs, mul_first=mul_first,
                          cout=cout),
        grid=(NTILES,),
        in_specs=in_specs,
        out_specs=[
            pl.BlockSpec((TILE, GW), lambda i: (i, 0)),
            pl.BlockSpec((TILE, cout), lambda i: (i, 0)),
        ],
        out_shape=[
            jax.ShapeDtypeStruct((NPAD, GW), GDTYPE),
            jax.ShapeDtypeStruct((NPAD, cout), jnp.float32),
        ],
    )(*args)


def _a2_body(*refs, n_a, n_b, cout_a, cout_b):
    pc_ref, w1ra_ref, b1a_ref, w1rb_ref, b1b_ref = refs[2 * (n_a + n_b):-3]
    a_ref, ca_ref, cb_ref = refs[-3:]
    pc = pc_ref[...]
    pa = jnp.dot(pc, w1ra_ref[...], precision=PREC,
                 preferred_element_type=jnp.float32)
    pb = jnp.dot(pc, w1rb_ref[...], precision=PREC,
                 preferred_element_type=jnp.float32)
    acc_a, pos = _pairs_acc(refs, 0, n_a, False)
    acc_b, _ = _pairs_acc(refs, pos, n_b, False)
    aa = acc_a + pa
    ab = acc_b + pb
    if cout_a < GW:
        aa = jnp.pad(aa, ((0, 0), (0, GW - cout_a)))
    if cout_b < GW:
        ab = jnp.pad(ab, ((0, 0), (0, GW - cout_b)))
    bits_a = jax.lax.bitcast_convert_type(
        aa.astype(jnp.bfloat16).astype(jnp.float32), jnp.uint32)
    bits_b = jax.lax.bitcast_convert_type(
        ab.astype(jnp.bfloat16).astype(jnp.float32), jnp.uint32)
    a_ref[...] = (bits_a & jnp.uint32(0xFFFF0000)) | (bits_b >> 16)
    ca_ref[...] = b1a_ref[...] - pa
    cb_ref[...] = b1b_ref[...] - pb


def _a2_kernel(pairs_a, sa, pairs_b, sb, pc16):
    cout_a = sa["b1"].shape[1]
    cout_b = sb["b1"].shape[1]
    in_specs = []
    args = []
    for tup in pairs_a + pairs_b:
        for arr in tup:
            if arr.shape[0] == NPAD:
                in_specs.append(
                    pl.BlockSpec((TILE, arr.shape[1]), lambda i: (i, 0)))
            else:
                in_specs.append(pl.BlockSpec(arr.shape, lambda i: (0, 0)))
            args.append(arr)
    in_specs.append(pl.BlockSpec((TILE, 16), lambda i: (i, 0)))
    args.append(pc16)
    for arr in (sa["w1r"], sa["b1"], sb["w1r"], sb["b1"]):
        in_specs.append(pl.BlockSpec(arr.shape, lambda i: (0, 0)))
        args.append(arr)
    return pl.pallas_call(
        functools.partial(_a2_body, n_a=len(pairs_a), n_b=len(pairs_b),
                          cout_a=cout_a, cout_b=cout_b),
        grid=(NTILES,),
        in_specs=in_specs,
        out_specs=[
            pl.BlockSpec((TILE, GW), lambda i: (i, 0)),
            pl.BlockSpec((TILE, cout_a), lambda i: (i, 0)),
            pl.BlockSpec((TILE, cout_b), lambda i: (i, 0)),
        ],
        out_shape=[
            jax.ShapeDtypeStruct((NPAD, GW), jnp.uint32),
            jax.ShapeDtypeStruct((NPAD, cout_a), jnp.float32),
            jax.ShapeDtypeStruct((NPAD, cout_b), jnp.float32),
        ],
    )(*args)


def _lrelu(x):
    return jnp.where(x >= 0, x, 0.1 * x)


def _mlp_body(g_ref, c_ref, w2_ref, b2_ref, w3_ref, b3_ref, *rest,
              act, has_res, cout, unpack):
    if has_res:
        res_ref, o_ref = rest
    else:
        (o_ref,) = rest
    c = c_ref[...]
    w2 = w2_ref[...]
    b2 = b2_ref[...]
    w3 = w3_ref[...]
    b3 = b3_ref[...]
    acc = None
    for k in range(K):
        g = g_ref[k]
        if unpack == "hi":
            g = jax.lax.bitcast_convert_type(
                g & jnp.uint32(0xFFFF0000), jnp.float32)
        elif unpack == "lo":
            g = jax.lax.bitcast_convert_type(g << 16, jnp.float32)
        h = _lrelu(g[:, :cout].astype(jnp.float32) + c)
        h = _lrelu(jnp.dot(h, w2, precision=PREC,
                           preferred_element_type=jnp.float32) + b2)
        h = _lrelu(jnp.dot(h, w3, precision=PREC,
                           preferred_element_type=jnp.float32) + b3)
        acc = h if acc is None else jnp.maximum(acc, h)
    if act == "sigmoid":
        acc = jax.nn.sigmoid(acc)
    elif act == "tanh":
        acc = jnp.tanh(acc)
    if has_res:
        acc = acc + res_ref[...]
    o_ref[...] = acc


def _mlp_max(g, cbias, p, act=None, res=None, lane=0, unpack=None):
    cout = cbias.shape[1]
    cout2 = p["w3"].shape[1]
    rows = g.shape[1]
    in_specs = [
        pl.BlockSpec((K, TILE, GW), lambda i, lane=lane: (0, i, lane)),
        pl.BlockSpec((TILE, cout), lambda i: (i, 0)),
        pl.BlockSpec(p["w2"].shape, lambda i: (0, 0)),
        pl.BlockSpec((1, cout2), lambda i: (0, 0)),
        pl.BlockSpec(p["w3"].shape, lambda i: (0, 0)),
        pl.BlockSpec((1, cout2), lambda i: (0, 0)),
    ]
    args = [g, cbias, p["w2"], p["b2"], p["w3"], p["b3"]]
    if res is not None:
        in_specs.append(pl.BlockSpec((TILE, cout2), lambda i: (i, 0)))
        args.append(res)
    return pl.pallas_call(
        functools.partial(_mlp_body, act=act, has_res=res is not None,
                          cout=cout, unpack=unpack),
        grid=(rows // TILE,),
        in_specs=in_specs,
        out_specs=pl.BlockSpec((TILE, cout2), lambda i: (i, 0)),
        out_shape=jax.ShapeDtypeStruct((rows, cout2), jnp.float32),
    )(*args)


def _gru_body(h_ref, z_ref, q_ref, o_ref):
    z = z_ref[...]
    o_ref[...] = (1.0 - z) * h_ref[...] + z * q_ref[...]


def _gru_update(h, z, q):
    d = h.shape[1]
    spec = pl.BlockSpec((TILE, d), lambda i: (i, 0))
    return pl.pallas_call(
        _gru_body,
        grid=(NTILES,),
        in_specs=[spec, spec, spec],
        out_specs=spec,
        out_shape=jax.ShapeDtypeStruct((NPAD, d), jnp.float32),
    )(h, z, q)


def _fc_body(f_ref, w_ref, b_ref, flow_ref, o_ref):
    o_ref[...] = (
        jnp.dot(f_ref[...], w_ref[...], precision=PREC_HI,
                preferred_element_type=jnp.float32)
        + b_ref[...]
        + flow_ref[...]
    )


def _fc_res(f, w, b, flow8):
    cin = f.shape[1]
    cout = w.shape[1]
    return pl.pallas_call(
        _fc_body,
        grid=(NTILES,),
        in_specs=[
            pl.BlockSpec((TILE, cin), lambda i: (i, 0)),
            pl.BlockSpec((cin, cout), lambda i: (0, 0)),
            pl.BlockSpec((1, cout), lambda i: (0, 0)),
            pl.BlockSpec((TILE, cout), lambda i: (i, 0)),
        ],
        out_specs=pl.BlockSpec((TILE, cout), lambda i: (i, 0)),
        out_shape=jax.ShapeDtypeStruct((NPAD, cout), jnp.float32),
    )(f, w, b, flow8)


def _split_setconv(p, cin):
    w1 = p["l1"]["W"]
    cout = w1.shape[1]
    w1r = jnp.zeros((16, cout), jnp.float32).at[:3].set(w1[cin:cin + 3])
    return {
        "w1s": w1[:cin],
        "w1r": w1r,
        "b1": p["l1"]["b"][None, :],
        "w2": p["l2"]["W"],
        "b2": p["l2"]["b"][None, :],
        "w3": p["l3"]["W"],
        "b3": p["l3"]["b"][None, :],
    }


def _pad_rows(x):
    return jnp.pad(x, ((0, NPAD - x.shape[0]), (0, 0)))


def kernel(pc, edges, flow, params):
    pc0 = pc[0]
    flow0 = flow[0]
    e = edges[0].astype(jnp.int32)

    et = jnp.pad(e.T, ((0, 0), (0, NPAD - N)))
    idxs = [et[:, c * CS:(c + 1) * CS].reshape(1, K * CS) for c in range(CHUNKS)]

    pc16 = jnp.pad(_pad_rows(pc0), ((0, 0), (0, 13)))
    flow8 = jnp.pad(_pad_rows(flow0), ((0, 0), (0, 5)))

    sp = {name: _split_setconv(params[name], cin) for name, cin in [
        ("feat_conv1", 3), ("feat_conv2", 32), ("feat_conv3", 64),
        ("h_conv1", 128), ("h_conv2", 128), ("delta_flow_conv_x", 128),
        ("flow_conv_x", 128), ("convz", 256), ("convr", 256), ("convq", 256),
        ("flow_conv1", 128), ("flow_conv2", 128),
    ]}

    def make_table(pairs, name, mul_first=False):
        s = sp[name]
        return _a_kernel(pairs, pc16, s["w1r"], s["b1"], mul_first=mul_first)

    def gather(table):
        return [_sc_gather(table, idxs[c]).reshape(K, CS, table.shape[1])
                for c in range(CHUNKS)]

    def mlp(gchunks, cb, name, act=None, res=None, unpack=None):
        outs = []
        for c, g in enumerate(gchunks):
            sl = slice(c * CS, (c + 1) * CS)
            outs.append(_mlp_max(
                g, cb[sl], sp[name], act=act,
                res=None if res is None else res[sl], unpack=unpack))
        return jnp.concatenate(outs, axis=0)

    def layer(pairs, name, act=None, res=None, mul_first=False):
        a, cb = make_table(pairs, name, mul_first=mul_first)
        return mlp(gather(a), cb, name, act=act, res=res)

    we = params["flow_encoder"]["W"]
    be = params["flow_encoder"]["b"]
    wfold = jnp.zeros((8, 128), jnp.float32).at[:3].set(
        jnp.dot(we, sp["flow_conv_x"]["w1s"], precision=PREC_HI))
    sp["flow_conv_x"]["b1"] = sp["flow_conv_x"]["b1"] + jnp.dot(
        be, sp["flow_conv_x"]["w1s"], precision=PREC_HI)[None, :]

    w1s_fc1_16 = jnp.zeros((16, 32), jnp.float32).at[:3].set(sp["feat_conv1"]["w1s"])
    a1, cb_fc1, cb_fx = _a2_kernel(
        [(pc16, w1s_fc1_16)], sp["feat_conv1"],
        [(flow8, wfold)], sp["flow_conv_x"], pc16)
    g1 = gather(a1)
    feat = mlp(g1, cb_fc1, "feat_conv1", unpack="hi")
    feat = layer([(feat, sp["feat_conv2"]["w1s"])], "feat_conv2")
    feat = layer([(feat, sp["feat_conv3"]["w1s"])], "feat_conv3")

    a4, cb_h1, cb_df = _a2_kernel(
        [(feat, sp["h_conv1"]["w1s"])], sp["h_conv1"],
        [(feat, sp["delta_flow_conv_x"]["w1s"])], sp["delta_flow_conv_x"], pc16)
    g4 = gather(a4)
    h = mlp(g4, cb_h1, "h_conv1", unpack="hi")
    corr = mlp(g4, cb_df, "delta_flow_conv_x", unpack="lo")
    h = layer([(h, sp["h_conv2"]["w1s"])], "h_conv2", act="tanh")
    x = mlp(g1, cb_fx, "flow_conv_x", res=corr, unpack="lo")

    a6, cb_z, cb_r = _a2_kernel(
        [(h, sp["convz"]["w1s"][:128]), (x, sp["convz"]["w1s"][128:])],
        sp["convz"],
        [(h, sp["convr"]["w1s"][:128]), (x, sp["convr"]["w1s"][128:])],
        sp["convr"], pc16)
    g6 = gather(a6)
    z = mlp(g6, cb_z, "convz", act="sigmoid", unpack="hi")
    r = mlp(g6, cb_r, "convr", act="sigmoid", unpack="lo")
    q = layer([(r, h, sp["convq"]["w1s"][:128]), (x, sp["convq"]["w1s"][128:])],
              "convq", act="tanh", mul_first=True)
    h = _gru_update(h, z, q)

    f = layer([(h, sp["flow_conv1"]["w1s"])], "flow_conv1")
    f = layer([(f, sp["flow_conv2"]["w1s"])], "flow_conv2")
    fcw = jnp.pad(params["fc"]["W"], ((0, 0), (0, 5)))
    fcb = jnp.pad(params["fc"]["b"], (0, 5))[None, :]
    out = _fc_res(f, fcw, fcb, flow8)

    return out[:N, :3][None]
